# baseline (speedup 1.0000x reference)
import functools

import jax
import jax.numpy as jnp
from jax import lax
from jax.experimental import pallas as pl
from jax.experimental.pallas import tpu as pltpu
from jax.experimental.pallas import tpu_sc as plsc

NSEG = 512
NROWS = 50000
HALF = 768
NV = HALF // 16
C = 40
Q = 3200
G = C // 8


def _zvec():
    return jnp.zeros((16,), jnp.float32)


def _body(h0, h1, h2, idx, out,
          buf0, buf1, iv, stage, t1, t2, shared, bound, sem0, sem1):
    c = lax.axis_index("c")
    s = lax.axis_index("s")

    def zrow(i, _):
        buf0[i // NV, pl.ds((i % NV) * 16, 16)] = _zvec()
        return 0
    lax.fori_loop(0, 32 * NV, zrow, 0)
    pltpu.sync_copy(buf0.at[pl.ds(0, 32), :], shared.at[pl.ds(s * 32, 32), :])
    plsc.subcore_barrier()

    r0 = s * Q
    nrows = jnp.minimum(Q, NROWS - r0)
    nch = nrows // C
    npairs = nch // 2

    @pl.when(s < 15)
    def _():
        pltpu.sync_copy(idx.at[pl.ds(r0, Q)], iv.at[pl.ds(0, Q)])

    @pl.when(s == 15)
    def _():
        pltpu.sync_copy(idx.at[pl.ds(r0, NROWS - 15 * Q)],
                        iv.at[pl.ds(0, NROWS - 15 * Q)])

    first_seg = iv[pl.ds(0, 16)][0]

    def dmas(buf, sem, r):
        ops0 = [(h0.at[pl.ds(r, C), :], buf.at[:, pl.ds(0, 512)], sem),
                (h1.at[pl.ds(r, C), pl.ds(0, 256)], buf.at[:, pl.ds(512, 256)], sem)]
        ops1 = [(h1.at[pl.ds(r, C), pl.ds(256, 256)], buf.at[:, pl.ds(0, 256)], sem),
                (h2.at[pl.ds(r, C), :], buf.at[:, pl.ds(256, 512)], sem)]
        return ops0, ops1

    def issue(buf, sem, r):
        return
        ops0, ops1 = dmas(buf, sem, r)

        @pl.when(c == 0)
        def _():
            for o in ops0:
                pltpu.async_copy(*o)

        @pl.when(c == 1)
        def _():
            for o in ops1:
                pltpu.async_copy(*o)

    def drain(buf, sem, r):
        return
        ops0, ops1 = dmas(buf, sem, r)

        @pl.when(c == 0)
        def _():
            for o in ops0:
                pltpu.make_async_copy(*o).wait()

        @pl.when(c == 1)
        def _():
            for o in ops1:
                pltpu.make_async_copy(*o).wait()

    def flush(seg, accs, slot):
        for j in range(NV):
            stage[pl.ds(j * 16, 16)] = accs[j]

        @pl.when(seg == first_seg)
        def _():
            pltpu.sync_copy(stage, bound.at[slot])

        @pl.when(seg != first_seg)
        def _():
            pltpu.sync_copy(stage, shared.at[seg])

    def compute(buf, ci, carry):
        def group(g, carry):
            cur = carry[0]
            accs = list(carry[1:])
            vseg = iv[pl.ds(ci * C + 8 * g, 16)]
            for k in range(8):
                row = 8 * g + k
                seg = vseg[k]
                changed = seg != cur

                @pl.when(changed)
                def _(cur=cur, accs=tuple(accs)):
                    flush(cur, accs, 2 * s)

                z = _zvec()
                for j in range(NV):
                    accs[j] = (jnp.where(changed, z, accs[j])
                               + buf[row, pl.ds(j * 16, 16)])
                cur = seg
            return (cur,) + tuple(accs)
        return lax.fori_loop(0, G, group, carry)

    issue(buf0, sem0, r0)

    def pair(t, carry):
        ra = r0 + (2 * t) * C
        rb = ra + C
        issue(buf1, sem1, rb)
        drain(buf0, sem0, ra)
        carry = compute(buf0, 2 * t, carry)

        @pl.when(2 * t + 2 < nch)
        def _():
            issue(buf0, sem0, rb + C)
        drain(buf1, sem1, rb)
        carry = compute(buf1, 2 * t + 1, carry)
        return carry

    init = (first_seg,) + tuple(_zvec() for _ in range(NV))
    carry = lax.fori_loop(0, npairs, pair, init)

    def tail(i, carry):
        drain(buf0, sem0, r0 + i * C)
        return compute(buf0, i, carry)

    carry = lax.fori_loop(2 * npairs, nch, tail, carry)
    cur = carry[0]
    accs = tuple(carry[1:])

    for j in range(NV):
        stage[pl.ds(j * 16, 16)] = accs[j]

    @pl.when(cur == first_seg)
    def _():
        pltpu.sync_copy(stage, bound.at[2 * s])

    @pl.when(cur != first_seg)
    def _():
        pltpu.sync_copy(stage, bound.at[2 * s + 1])

    plsc.subcore_barrier()

    @pl.when(s == 0)
    def _():
        def addslot(slot, seg):
            pltpu.sync_copy(shared.at[seg], t1)
            pltpu.sync_copy(bound.at[slot], t2)
            for j in range(NV):
                t1[pl.ds(j * 16, 16)] = (t1[pl.ds(j * 16, 16)]
                                         + t2[pl.ds(j * 16, 16)])
            pltpu.sync_copy(t1, shared.at[seg])

        def perw(w, _):
            beg = pl.multiple_of(w * Q, 8)
            pltpu.sync_copy(idx.at[pl.ds(beg, 16)], iv.at[pl.ds(0, 16)])
            fs = iv[pl.ds(0, 16)][0]
            end = pl.multiple_of(jnp.minimum(w * Q + Q, NROWS) - 16, 8)
            pltpu.sync_copy(idx.at[pl.ds(end, 16)], iv.at[pl.ds(0, 16)])
            ls = iv[pl.ds(0, 16)][15]
            addslot(2 * w, fs)

            @pl.when(ls != fs)
            def _():
                addslot(2 * w + 1, ls)
            return 0
        lax.fori_loop(0, 16, perw, 0)

    plsc.subcore_barrier()

    @pl.when(c == 0)
    def _():
        pltpu.sync_copy(shared.at[pl.ds(s * 32, 32), :],
                        out.at[pl.ds(s * 32, 32), pl.ds(0, HALF)])

    @pl.when(c == 1)
    def _():
        pltpu.sync_copy(shared.at[pl.ds(s * 32, 32), :],
                        out.at[pl.ds(s * 32, 32), pl.ds(HALF, HALF)])


@jax.jit
def kernel(h0, h1, h2, index):
    k = pl.kernel(
        _body,
        out_type=jax.ShapeDtypeStruct((NSEG, 3 * 512), jnp.float32),
        mesh=plsc.VectorSubcoreMesh(core_axis_name="c", subcore_axis_name="s"),
        scratch_types=[
            pltpu.VMEM((C, HALF), jnp.float32),
            pltpu.VMEM((C, HALF), jnp.float32),
            pltpu.VMEM((Q + 16,), jnp.int32),
            pltpu.VMEM((HALF,), jnp.float32),
            pltpu.VMEM((HALF,), jnp.float32),
            pltpu.VMEM((HALF,), jnp.float32),
            pltpu.VMEM_SHARED((NSEG, HALF), jnp.float32),
            pltpu.VMEM_SHARED((32, HALF), jnp.float32),
            pltpu.SemaphoreType.DMA,
            pltpu.SemaphoreType.DMA,
        ],
    )
    return k(h0, h1, h2, index)


# speedup vs baseline: 1.5350x; 1.5350x over previous
import functools

import jax
import jax.numpy as jnp
from jax import lax
from jax.experimental import pallas as pl
from jax.experimental.pallas import tpu as pltpu
from jax.experimental.pallas import tpu_sc as plsc

NSEG = 512
NROWS = 50000
HALF = 768
NV = HALF // 16
C = 64
Q = 3200
G = C // 16


def _zvec():
    return jnp.zeros((16,), jnp.float32)


def _body(h0, h1, h2, idx, out,
          buf0, buf1, iv, areg, t1, t2, shared, bound, sem0, sem1):
    c = lax.axis_index("c")
    s = lax.axis_index("s")

    def zrow(i, _):
        buf0[i // NV, pl.ds((i % NV) * 16, 16)] = _zvec()
        return 0
    lax.fori_loop(0, 32 * NV, zrow, 0)
    pltpu.sync_copy(buf0.at[pl.ds(0, 32), :], shared.at[pl.ds(s * 32, 32), :])
    plsc.subcore_barrier()

    for j in range(NV):
        areg[pl.ds(j * 16, 16)] = _zvec()

    r0 = s * Q
    nrows = jnp.minimum(Q, NROWS - r0)
    nch = nrows // C
    npairs = nch // 2

    @pl.when(s < 15)
    def _():
        pltpu.sync_copy(idx.at[pl.ds(r0, Q)], iv.at[pl.ds(0, Q)])

    @pl.when(s == 15)
    def _():
        pltpu.sync_copy(idx.at[pl.ds(r0, NROWS - 15 * Q)],
                        iv.at[pl.ds(0, NROWS - 15 * Q)])

    first_seg = iv[pl.ds(0, 16)][0]

    def dmas(buf, sem, r, n):
        ops0 = [(h0.at[pl.ds(r, n), :], buf.at[pl.ds(0, n), pl.ds(0, 512)], sem),
                (h1.at[pl.ds(r, n), pl.ds(0, 256)], buf.at[pl.ds(0, n), pl.ds(512, 256)], sem)]
        ops1 = [(h1.at[pl.ds(r, n), pl.ds(256, 256)], buf.at[pl.ds(0, n), pl.ds(0, 256)], sem),
                (h2.at[pl.ds(r, n), :], buf.at[pl.ds(0, n), pl.ds(256, 512)], sem)]
        return ops0, ops1

    def issue(buf, sem, r):
        ops0, ops1 = dmas(buf, sem, r, C)

        @pl.when(c == 0)
        def _():
            for o in ops0:
                pltpu.async_copy(*o)

        @pl.when(c == 1)
        def _():
            for o in ops1:
                pltpu.async_copy(*o)

    def drain(buf, sem, r):
        ops0, ops1 = dmas(buf, sem, r, C)

        @pl.when(c == 0)
        def _():
            for o in ops0:
                pltpu.make_async_copy(*o).wait()

        @pl.when(c == 1)
        def _():
            for o in ops1:
                pltpu.make_async_copy(*o).wait()

    NH = NV // 2

    def group16h(buf, base, off, cur, half):
        cb = half * 384
        v = iv[pl.ds(off, 16)]
        e0 = v[0]
        e15 = v[15]
        fast = jnp.logical_and(e0 == e15, e0 == cur)

        @pl.when(fast)
        def _():
            def frow(k, accs):
                return tuple(accs[j] + buf[base + k, pl.ds(cb + j * 16, 16)]
                             for j in range(NH))
            accs = tuple(areg[pl.ds(cb + j * 16, 16)] for j in range(NH))
            accs = lax.fori_loop(0, 16, frow, accs)
            for j in range(NH):
                areg[pl.ds(cb + j * 16, 16)] = accs[j]

        @pl.when(jnp.logical_not(fast))
        def _():
            def srow(k, carry):
                lcur = carry[0]
                accs = list(carry[1:])
                seg = iv[pl.ds(off + k, 16)][0]
                changed = seg != lcur

                @pl.when(changed)
                def _(lcur=lcur, accs=tuple(accs)):
                    for j in range(NH):
                        areg[pl.ds(cb + j * 16, 16)] = accs[j]

                    @pl.when(lcur == first_seg)
                    def _():
                        pltpu.sync_copy(areg.at[pl.ds(cb, 384)],
                                        bound.at[2 * s, pl.ds(cb, 384)])

                    @pl.when(lcur != first_seg)
                    def _():
                        pltpu.sync_copy(areg.at[pl.ds(cb, 384)],
                                        shared.at[lcur, pl.ds(cb, 384)])

                z = _zvec()
                for j in range(NH):
                    accs[j] = (jnp.where(changed, z, accs[j])
                               + buf[base + k, pl.ds(cb + j * 16, 16)])
                return (seg,) + tuple(accs)

            accs = tuple(areg[pl.ds(cb + j * 16, 16)] for j in range(NH))
            carry = lax.fori_loop(0, 16, srow, (cur,) + accs)
            for j in range(NH):
                areg[pl.ds(cb + j * 16, 16)] = carry[j + 1]

        return e15

    def group16(buf, base, off, cur):
        group16h(buf, base, off, cur, 0)
        return group16h(buf, base, off, cur, 1)

    def compute(buf, ci, cur):
        def group(g, cur):
            return group16(buf, 16 * g, ci * C + 16 * g, cur)
        return lax.fori_loop(0, G, group, cur)

    issue(buf0, sem0, r0)

    def pair(t, cur):
        ra = r0 + (2 * t) * C
        rb = ra + C
        issue(buf1, sem1, rb)
        drain(buf0, sem0, ra)
        cur = compute(buf0, 2 * t, cur)

        @pl.when(2 * t + 2 < nch)
        def _():
            issue(buf0, sem0, rb + C)
        drain(buf1, sem1, rb)
        cur = compute(buf1, 2 * t + 1, cur)
        return cur

    cur = lax.fori_loop(0, npairs, pair, first_seg)

    def tail(i, cur):
        drain(buf0, sem0, r0 + i * C)
        return compute(buf0, i, cur)

    cur = lax.fori_loop(2 * npairs, nch, tail, cur)

    def rem16(i, cur):
        r = r0 + nch * C + 16 * i
        ops0, ops1 = dmas(buf0, sem0, r, 16)

        @pl.when(c == 0)
        def _():
            for o in ops0:
                pltpu.sync_copy(o[0], o[1])

        @pl.when(c == 1)
        def _():
            for o in ops1:
                pltpu.sync_copy(o[0], o[1])
        return group16(buf0, 0, nch * C + 16 * i, cur)

    cur = lax.fori_loop(0, (nrows - nch * C) // 16, rem16, cur)

    @pl.when(cur == first_seg)
    def _():
        pltpu.sync_copy(areg, bound.at[2 * s])

    @pl.when(cur != first_seg)
    def _():
        pltpu.sync_copy(areg, bound.at[2 * s + 1])

    plsc.subcore_barrier()

    @pl.when(s == 0)
    def _():
        def addslot(slot, seg):
            pltpu.sync_copy(shared.at[seg], t1)
            pltpu.sync_copy(bound.at[slot], t2)
            for j in range(NV):
                t1[pl.ds(j * 16, 16)] = (t1[pl.ds(j * 16, 16)]
                                         + t2[pl.ds(j * 16, 16)])
            pltpu.sync_copy(t1, shared.at[seg])

        def perw(w, _):
            beg = pl.multiple_of(w * Q, 8)
            pltpu.sync_copy(idx.at[pl.ds(beg, 16)], iv.at[pl.ds(0, 16)])
            fs = iv[pl.ds(0, 16)][0]
            end = pl.multiple_of(jnp.minimum(w * Q + Q, NROWS) - 16, 8)
            pltpu.sync_copy(idx.at[pl.ds(end, 16)], iv.at[pl.ds(0, 16)])
            ls = iv[pl.ds(0, 16)][15]
            addslot(2 * w, fs)

            @pl.when(ls != fs)
            def _():
                addslot(2 * w + 1, ls)
            return 0
        lax.fori_loop(0, 16, perw, 0)

    plsc.subcore_barrier()

    @pl.when(c == 0)
    def _():
        pltpu.sync_copy(shared.at[pl.ds(s * 32, 32), :],
                        out.at[pl.ds(s * 32, 32), pl.ds(0, HALF)])

    @pl.when(c == 1)
    def _():
        pltpu.sync_copy(shared.at[pl.ds(s * 32, 32), :],
                        out.at[pl.ds(s * 32, 32), pl.ds(HALF, HALF)])


@jax.jit
def kernel(h0, h1, h2, index):
    k = pl.kernel(
        _body,
        out_type=jax.ShapeDtypeStruct((NSEG, 3 * 512), jnp.float32),
        mesh=plsc.VectorSubcoreMesh(core_axis_name="c", subcore_axis_name="s"),
        scratch_types=[
            pltpu.VMEM((C, HALF), jnp.float32),
            pltpu.VMEM((C, HALF), jnp.float32),
            pltpu.VMEM((Q + 16,), jnp.int32),
            pltpu.VMEM((HALF,), jnp.float32),
            pltpu.VMEM((HALF,), jnp.float32),
            pltpu.VMEM((HALF,), jnp.float32),
            pltpu.VMEM_SHARED((NSEG, HALF), jnp.float32),
            pltpu.VMEM_SHARED((32, HALF), jnp.float32),
            pltpu.SemaphoreType.DMA,
            pltpu.SemaphoreType.DMA,
        ],
    )
    return k(h0, h1, h2, index)


# speedup vs baseline: 2.0021x; 1.3043x over previous
import functools

import jax
import jax.numpy as jnp
from jax import lax
from jax.experimental import pallas as pl
from jax.experimental.pallas import tpu as pltpu
from jax.experimental.pallas import tpu_sc as plsc

NSEG = 512
NROWS = 50000
HALF = 512
NV = HALF // 16
C = 80
Q = 3200
G = C // 16


def _zvec():
    return jnp.zeros((16,), jnp.float32)


def _body(h0, h1, idx, out,
          buf0, buf1, iv, areg, t1, t2, shared, bound, sem0, sem1):
    c = lax.axis_index("c")
    s = lax.axis_index("s")

    def zrow(i, _):
        buf0[i // NV, pl.ds((i % NV) * 16, 16)] = _zvec()
        return 0
    lax.fori_loop(0, 32 * NV, zrow, 0)
    pltpu.sync_copy(buf0.at[pl.ds(0, 32), :], shared.at[pl.ds(s * 32, 32), :])
    plsc.subcore_barrier()

    for j in range(NV):
        areg[pl.ds(j * 16, 16)] = _zvec()

    r0 = s * Q
    nrows = jnp.minimum(Q, NROWS - r0)
    nch = nrows // C
    npairs = nch // 2

    @pl.when(s < 15)
    def _():
        pltpu.sync_copy(idx.at[pl.ds(r0, Q)], iv.at[pl.ds(0, Q)])

    @pl.when(s == 15)
    def _():
        pltpu.sync_copy(idx.at[pl.ds(r0, NROWS - 15 * Q)],
                        iv.at[pl.ds(0, NROWS - 15 * Q)])

    first_seg = iv[pl.ds(0, 16)][0]

    def dmas(buf, sem, r, n):
        ops0 = [(h0.at[pl.ds(r, n), :], buf.at[pl.ds(0, n), :], sem)]
        ops1 = [(h1.at[pl.ds(r, n), pl.ds(0, 256)], buf.at[pl.ds(0, n), pl.ds(0, 256)], sem),
                (h1.at[pl.ds(r, n), pl.ds(256, 256)], buf.at[pl.ds(0, n), pl.ds(256, 256)], sem)]
        return ops0, ops1

    def issue(buf, sem, r):
        ops0, ops1 = dmas(buf, sem, r, C)

        @pl.when(c == 0)
        def _():
            for o in ops0:
                pltpu.async_copy(*o)

        @pl.when(c == 1)
        def _():
            for o in ops1:
                pltpu.async_copy(*o)

    def drain(buf, sem, r):
        ops0, ops1 = dmas(buf, sem, r, C)

        @pl.when(c == 0)
        def _():
            for o in ops0:
                pltpu.make_async_copy(*o).wait()

        @pl.when(c == 1)
        def _():
            for o in ops1:
                pltpu.make_async_copy(*o).wait()

    NH = NV // 2

    def group16h(buf, base, off, cur, half):
        cb = half * 256
        v = iv[pl.ds(off, 16)]
        e0 = v[0]
        e15 = v[15]
        fast = jnp.logical_and(e0 == e15, e0 == cur)

        @pl.when(fast)
        def _():
            def frow(k, accs):
                return tuple(accs[j] + buf[base + k, pl.ds(cb + j * 16, 16)]
                             for j in range(NH))
            accs = tuple(areg[pl.ds(cb + j * 16, 16)] for j in range(NH))
            accs = lax.fori_loop(0, 16, frow, accs)
            for j in range(NH):
                areg[pl.ds(cb + j * 16, 16)] = accs[j]

        @pl.when(jnp.logical_not(fast))
        def _():
            def srow(k, carry):
                lcur = carry[0]
                accs = list(carry[1:])
                seg = iv[pl.ds(off + k, 16)][0]
                changed = seg != lcur

                @pl.when(changed)
                def _(lcur=lcur, accs=tuple(accs)):
                    for j in range(NH):
                        areg[pl.ds(cb + j * 16, 16)] = accs[j]

                    @pl.when(lcur == first_seg)
                    def _():
                        pltpu.sync_copy(areg.at[pl.ds(cb, 256)],
                                        bound.at[2 * s, pl.ds(cb, 256)])

                    @pl.when(lcur != first_seg)
                    def _():
                        pltpu.sync_copy(areg.at[pl.ds(cb, 256)],
                                        shared.at[lcur, pl.ds(cb, 256)])

                z = _zvec()
                for j in range(NH):
                    accs[j] = (jnp.where(changed, z, accs[j])
                               + buf[base + k, pl.ds(cb + j * 16, 16)])
                return (seg,) + tuple(accs)

            accs = tuple(areg[pl.ds(cb + j * 16, 16)] for j in range(NH))
            carry = lax.fori_loop(0, 16, srow, (cur,) + accs)
            for j in range(NH):
                areg[pl.ds(cb + j * 16, 16)] = carry[j + 1]

        return e15

    def group16(buf, base, off, cur):
        group16h(buf, base, off, cur, 0)
        return group16h(buf, base, off, cur, 1)

    def compute(buf, ci, cur):
        def group(g, cur):
            return group16(buf, 16 * g, ci * C + 16 * g, cur)
        return lax.fori_loop(0, G, group, cur)

    issue(buf0, sem0, r0)

    def pair(t, cur):
        ra = r0 + (2 * t) * C
        rb = ra + C
        issue(buf1, sem1, rb)
        drain(buf0, sem0, ra)
        cur = compute(buf0, 2 * t, cur)

        @pl.when(2 * t + 2 < nch)
        def _():
            issue(buf0, sem0, rb + C)
        drain(buf1, sem1, rb)
        cur = compute(buf1, 2 * t + 1, cur)
        return cur

    cur = lax.fori_loop(0, npairs, pair, first_seg)

    def tail(i, cur):
        drain(buf0, sem0, r0 + i * C)
        return compute(buf0, i, cur)

    cur = lax.fori_loop(2 * npairs, nch, tail, cur)

    def rem16(i, cur):
        r = r0 + nch * C + 16 * i
        ops0, ops1 = dmas(buf0, sem0, r, 16)

        @pl.when(c == 0)
        def _():
            for o in ops0:
                pltpu.sync_copy(o[0], o[1])

        @pl.when(c == 1)
        def _():
            for o in ops1:
                pltpu.sync_copy(o[0], o[1])
        return group16(buf0, 0, nch * C + 16 * i, cur)

    cur = lax.fori_loop(0, (nrows - nch * C) // 16, rem16, cur)

    @pl.when(cur == first_seg)
    def _():
        pltpu.sync_copy(areg, bound.at[2 * s])

    @pl.when(cur != first_seg)
    def _():
        pltpu.sync_copy(areg, bound.at[2 * s + 1])

    plsc.subcore_barrier()

    @pl.when(s == 0)
    def _():
        def addslot(slot, seg):
            pltpu.sync_copy(shared.at[seg], t1)
            pltpu.sync_copy(bound.at[slot], t2)
            for j in range(NV):
                t1[pl.ds(j * 16, 16)] = (t1[pl.ds(j * 16, 16)]
                                         + t2[pl.ds(j * 16, 16)])
            pltpu.sync_copy(t1, shared.at[seg])

        def perw(w, _):
            beg = pl.multiple_of(w * Q, 8)
            pltpu.sync_copy(idx.at[pl.ds(beg, 16)], iv.at[pl.ds(0, 16)])
            fs = iv[pl.ds(0, 16)][0]
            end = pl.multiple_of(jnp.minimum(w * Q + Q, NROWS) - 16, 8)
            pltpu.sync_copy(idx.at[pl.ds(end, 16)], iv.at[pl.ds(0, 16)])
            ls = iv[pl.ds(0, 16)][15]
            addslot(2 * w, fs)

            @pl.when(ls != fs)
            def _():
                addslot(2 * w + 1, ls)
            return 0
        lax.fori_loop(0, 16, perw, 0)

    plsc.subcore_barrier()

    @pl.when(c == 0)
    def _():
        pltpu.sync_copy(shared.at[pl.ds(s * 32, 32), :],
                        out.at[pl.ds(s * 32, 32), pl.ds(0, HALF)])

    @pl.when(c == 1)
    def _():
        pltpu.sync_copy(shared.at[pl.ds(s * 32, 32), :],
                        out.at[pl.ds(s * 32, 32), pl.ds(HALF, HALF)])


def _tc_body(idx_ref, h_ref, o_ref):
    i = pl.program_id(0)
    idxb = idx_ref[0, 0, :]
    rows = lax.broadcasted_iota(jnp.int32, (NSEG, 512), 1) + i * 512
    segs = lax.broadcasted_iota(jnp.int32, (NSEG, 512), 0)
    oh = jnp.where((idxb[None, :] == segs) & (rows < NROWS), 1.0, 0.0)
    contrib = jnp.dot(oh.astype(jnp.float32), h_ref[...],
                      preferred_element_type=jnp.float32)

    @pl.when(i == 0)
    def _():
        o_ref[...] = contrib

    @pl.when(i > 0)
    def _():
        o_ref[...] = o_ref[...] + contrib


_NB = (NROWS + 511) // 512


def _tc_matmul(idx3, h2):
    return pl.pallas_call(
        _tc_body,
        grid=(_NB,),
        in_specs=[pl.BlockSpec((1, 1, 512), lambda i: (i, 0, 0)),
                  pl.BlockSpec((512, 512), lambda i: (i, 0))],
        out_specs=pl.BlockSpec((NSEG, 512), lambda i: (0, 0)),
        out_shape=jax.ShapeDtypeStruct((NSEG, 512), jnp.float32),
    )(idx3, h2)


@jax.jit
def kernel(h0, h1, h2, index):
    k = pl.kernel(
        _body,
        out_type=jax.ShapeDtypeStruct((NSEG, 2 * HALF), jnp.float32),
        mesh=plsc.VectorSubcoreMesh(core_axis_name="c", subcore_axis_name="s"),
        scratch_types=[
            pltpu.VMEM((C, HALF), jnp.float32),
            pltpu.VMEM((C, HALF), jnp.float32),
            pltpu.VMEM((Q + 16,), jnp.int32),
            pltpu.VMEM((HALF,), jnp.float32),
            pltpu.VMEM((HALF,), jnp.float32),
            pltpu.VMEM((HALF,), jnp.float32),
            pltpu.VMEM_SHARED((NSEG, HALF), jnp.float32),
            pltpu.VMEM_SHARED((32, HALF), jnp.float32),
            pltpu.SemaphoreType.DMA,
            pltpu.SemaphoreType.DMA,
        ],
    )
    out01 = k(h0, h1, index)
    idx3 = jnp.pad(index, (0, _NB * 512 - NROWS)).reshape(_NB, 1, 512)
    out2 = _tc_matmul(idx3, h2)
    return jnp.concatenate([out01, out2], axis=1)


# speedup vs baseline: 2.1721x; 1.0849x over previous
import functools

import jax
import jax.numpy as jnp
from jax import lax
from jax.experimental import pallas as pl
from jax.experimental.pallas import tpu as pltpu
from jax.experimental.pallas import tpu_sc as plsc

NSEG = 512
NROWS = 50000
HALF = 512
NV = HALF // 16
C = 80
Q = 3200
G = C // 16


def _zvec():
    return jnp.zeros((16,), jnp.float32)


def _body(h0, h1, idx, bseg, out,
          buf0, buf1, iv, areg, t1, t2, shared, bound, sem0, sem1):
    c = lax.axis_index("c")
    s = lax.axis_index("s")

    r0 = s * Q
    nrows = jnp.minimum(Q, NROWS - r0)
    nch = nrows // C
    npairs = nch // 2

    def dmas(buf, sem, r, n):
        ops0 = [(h0.at[pl.ds(r, n), :], buf.at[pl.ds(0, n), :], sem)]
        ops1 = [(h1.at[pl.ds(r, n), pl.ds(0, 256)], buf.at[pl.ds(0, n), pl.ds(0, 256)], sem),
                (h1.at[pl.ds(r, n), pl.ds(256, 256)], buf.at[pl.ds(0, n), pl.ds(256, 256)], sem)]
        return ops0, ops1

    def issue(buf, sem, r):
        ops0, ops1 = dmas(buf, sem, r, C)

        @pl.when(c == 0)
        def _():
            for o in ops0:
                pltpu.async_copy(*o)

        @pl.when(c == 1)
        def _():
            for o in ops1:
                pltpu.async_copy(*o)

    def drain(buf, sem, r):
        ops0, ops1 = dmas(buf, sem, r, C)

        @pl.when(c == 0)
        def _():
            for o in ops0:
                pltpu.make_async_copy(*o).wait()

        @pl.when(c == 1)
        def _():
            for o in ops1:
                pltpu.make_async_copy(*o).wait()

    NH = NV // 2

    def group16h(buf, base, off, cur, half):
        cb = half * 256
        v = iv[pl.ds(off, 16)]
        e0 = v[0]
        e15 = v[15]
        fast = jnp.logical_and(e0 == e15, e0 == cur)

        @pl.when(fast)
        def _():
            def frow(k, accs):
                return tuple(accs[j] + buf[base + k, pl.ds(cb + j * 16, 16)]
                             for j in range(NH))
            accs = tuple(areg[pl.ds(cb + j * 16, 16)] for j in range(NH))
            accs = lax.fori_loop(0, 16, frow, accs)
            for j in range(NH):
                areg[pl.ds(cb + j * 16, 16)] = accs[j]

        @pl.when(jnp.logical_not(fast))
        def _():
            def srow(k, carry):
                lcur = carry[0]
                accs = list(carry[1:])
                seg = iv[pl.ds(off + k, 16)][0]
                changed = seg != lcur

                @pl.when(changed)
                def _(lcur=lcur, accs=tuple(accs)):
                    for j in range(NH):
                        areg[pl.ds(cb + j * 16, 16)] = accs[j]

                    @pl.when(lcur == first_seg)
                    def _():
                        pltpu.sync_copy(areg.at[pl.ds(cb, 256)],
                                        bound.at[2 * s, pl.ds(cb, 256)])

                    @pl.when(lcur != first_seg)
                    def _():
                        pltpu.sync_copy(areg.at[pl.ds(cb, 256)],
                                        shared.at[lcur, pl.ds(cb, 256)])

                z = _zvec()
                for j in range(NH):
                    accs[j] = (jnp.where(changed, z, accs[j])
                               + buf[base + k, pl.ds(cb + j * 16, 16)])
                return (seg,) + tuple(accs)

            accs = tuple(areg[pl.ds(cb + j * 16, 16)] for j in range(NH))
            carry = lax.fori_loop(0, 16, srow, (cur,) + accs)
            for j in range(NH):
                areg[pl.ds(cb + j * 16, 16)] = carry[j + 1]

        return e15

    def group16(buf, base, off, cur):
        group16h(buf, base, off, cur, 0)
        return group16h(buf, base, off, cur, 1)

    def compute(buf, ci, cur):
        def group(g, cur):
            return group16(buf, 16 * g, ci * C + 16 * g, cur)
        return lax.fori_loop(0, G, group, cur)

    issue(buf0, sem0, r0)

    @pl.when(s < 15)
    def _():
        pltpu.sync_copy(idx.at[pl.ds(r0, Q)], iv.at[pl.ds(0, Q)])

    @pl.when(s == 15)
    def _():
        pltpu.sync_copy(idx.at[pl.ds(r0, NROWS - 15 * Q)],
                        iv.at[pl.ds(0, NROWS - 15 * Q)])

    first_seg = iv[pl.ds(0, 16)][0]

    def zrow(i, _):
        buf1[i // NV, pl.ds((i % NV) * 16, 16)] = _zvec()
        return 0
    lax.fori_loop(0, 32 * NV, zrow, 0)
    pltpu.sync_copy(buf1.at[pl.ds(0, 32), :], shared.at[pl.ds(s * 32, 32), :])

    for j in range(NV):
        areg[pl.ds(j * 16, 16)] = _zvec()
    plsc.subcore_barrier()

    def pair(t, cur):
        ra = r0 + (2 * t) * C
        rb = ra + C
        issue(buf1, sem1, rb)
        drain(buf0, sem0, ra)
        cur = compute(buf0, 2 * t, cur)

        @pl.when(2 * t + 2 < nch)
        def _():
            issue(buf0, sem0, rb + C)
        drain(buf1, sem1, rb)
        cur = compute(buf1, 2 * t + 1, cur)
        return cur

    cur = lax.fori_loop(0, npairs, pair, first_seg)

    def tail(i, cur):
        drain(buf0, sem0, r0 + i * C)
        return compute(buf0, i, cur)

    cur = lax.fori_loop(2 * npairs, nch, tail, cur)

    def rem16(i, cur):
        r = r0 + nch * C + 16 * i
        ops0, ops1 = dmas(buf0, sem0, r, 16)

        @pl.when(c == 0)
        def _():
            for o in ops0:
                pltpu.sync_copy(o[0], o[1])

        @pl.when(c == 1)
        def _():
            for o in ops1:
                pltpu.sync_copy(o[0], o[1])
        return group16(buf0, 0, nch * C + 16 * i, cur)

    cur = lax.fori_loop(0, (nrows - nch * C) // 16, rem16, cur)

    @pl.when(cur == first_seg)
    def _():
        pltpu.sync_copy(areg, bound.at[2 * s])

    @pl.when(cur != first_seg)
    def _():
        pltpu.sync_copy(areg, bound.at[2 * s + 1])

    plsc.subcore_barrier()

    @pl.when(s == 0)
    def _():
        def addslot(slot, seg):
            pltpu.async_copy(shared.at[seg], t1, sem0)
            pltpu.async_copy(bound.at[slot], t2, sem1)
            pltpu.make_async_copy(shared.at[seg], t1, sem0).wait()
            pltpu.make_async_copy(bound.at[slot], t2, sem1).wait()
            for j in range(NV):
                t1[pl.ds(j * 16, 16)] = (t1[pl.ds(j * 16, 16)]
                                         + t2[pl.ds(j * 16, 16)])
            pltpu.sync_copy(t1, shared.at[seg])

        pltpu.sync_copy(bseg, iv.at[pl.ds(0, 32)])
        vfs = iv[pl.ds(0, 16)]
        vls = iv[pl.ds(16, 16)]
        for w in range(16):
            fs = vfs[w]
            ls = vls[w]
            addslot(2 * w, fs)

            @pl.when(ls != fs)
            def _():
                addslot(2 * w + 1, ls)

    plsc.subcore_barrier()

    @pl.when(c == 0)
    def _():
        pltpu.sync_copy(shared.at[pl.ds(s * 32, 32), :],
                        out.at[pl.ds(s * 32, 32), pl.ds(0, HALF)])

    @pl.when(c == 1)
    def _():
        pltpu.sync_copy(shared.at[pl.ds(s * 32, 32), :],
                        out.at[pl.ds(s * 32, 32), pl.ds(HALF, HALF)])


def _tc_body(idx_ref, h_ref, o_ref):
    i = pl.program_id(0)
    idxb = idx_ref[0, 0, :]
    rows = lax.broadcasted_iota(jnp.int32, (NSEG, 512), 1) + i * 512
    segs = lax.broadcasted_iota(jnp.int32, (NSEG, 512), 0)
    oh = jnp.where((idxb[None, :] == segs) & (rows < NROWS), 1.0, 0.0)
    contrib = jnp.dot(oh.astype(jnp.float32), h_ref[...],
                      preferred_element_type=jnp.float32)

    @pl.when(i == 0)
    def _():
        o_ref[...] = contrib

    @pl.when(i > 0)
    def _():
        o_ref[...] = o_ref[...] + contrib


_NB = (NROWS + 511) // 512


def _tc_matmul(idx3, h2):
    return pl.pallas_call(
        _tc_body,
        grid=(_NB,),
        in_specs=[pl.BlockSpec((1, 1, 512), lambda i: (i, 0, 0)),
                  pl.BlockSpec((512, 512), lambda i: (i, 0))],
        out_specs=pl.BlockSpec((NSEG, 512), lambda i: (0, 0)),
        out_shape=jax.ShapeDtypeStruct((NSEG, 512), jnp.float32),
    )(idx3, h2)


@jax.jit
def kernel(h0, h1, h2, index):
    k = pl.kernel(
        _body,
        out_type=jax.ShapeDtypeStruct((NSEG, 2 * HALF), jnp.float32),
        mesh=plsc.VectorSubcoreMesh(core_axis_name="c", subcore_axis_name="s"),
        scratch_types=[
            pltpu.VMEM((C, HALF), jnp.float32),
            pltpu.VMEM((C, HALF), jnp.float32),
            pltpu.VMEM((Q + 16,), jnp.int32),
            pltpu.VMEM((HALF,), jnp.float32),
            pltpu.VMEM((HALF,), jnp.float32),
            pltpu.VMEM((HALF,), jnp.float32),
            pltpu.VMEM_SHARED((NSEG, HALF), jnp.float32),
            pltpu.VMEM_SHARED((32, HALF), jnp.float32),
            pltpu.SemaphoreType.DMA,
            pltpu.SemaphoreType.DMA,
        ],
    )
    starts = jnp.arange(16, dtype=jnp.int32) * Q
    ends = jnp.minimum(starts + Q, NROWS) - 1
    bseg = jnp.concatenate([index[starts], index[ends]])
    out01 = k(h0, h1, index, bseg)
    idx3 = jnp.pad(index, (0, _NB * 512 - NROWS)).reshape(_NB, 1, 512)
    out2 = _tc_matmul(idx3, h2)
    return jnp.concatenate([out01, out2], axis=1)


# speedup vs baseline: 2.1723x; 1.0001x over previous
import jax
import jax.numpy as jnp
from jax import lax
from jax.experimental import pallas as pl
from jax.experimental.pallas import tpu as pltpu
from jax.experimental.pallas import tpu_sc as plsc

NSEG = 512
NROWS = 50000
HALF = 512
NV = HALF // 16
C = 80
Q = 3200
G = C // 16


def _zvec():
    return jnp.zeros((16,), jnp.float32)


def _body(h0, h1, idx, bseg, out,
          buf0, buf1, iv, areg, t1, t2, shared, bound, sem0, sem1):
    c = lax.axis_index("c")
    s = lax.axis_index("s")

    r0 = s * Q
    nrows = jnp.minimum(Q, NROWS - r0)
    nch = nrows // C
    npairs = nch // 2

    def dmas(buf, sem, r, n):
        ops0 = [(h0.at[pl.ds(r, n), :], buf.at[pl.ds(0, n), :], sem)]
        ops1 = [(h1.at[pl.ds(r, n), pl.ds(0, 256)], buf.at[pl.ds(0, n), pl.ds(0, 256)], sem),
                (h1.at[pl.ds(r, n), pl.ds(256, 256)], buf.at[pl.ds(0, n), pl.ds(256, 256)], sem)]
        return ops0, ops1

    def issue(buf, sem, r):
        ops0, ops1 = dmas(buf, sem, r, C)

        @pl.when(c == 0)
        def _():
            for o in ops0:
                pltpu.async_copy(*o)

        @pl.when(c == 1)
        def _():
            for o in ops1:
                pltpu.async_copy(*o)

    def drain(buf, sem, r):
        ops0, ops1 = dmas(buf, sem, r, C)

        @pl.when(c == 0)
        def _():
            for o in ops0:
                pltpu.make_async_copy(*o).wait()

        @pl.when(c == 1)
        def _():
            for o in ops1:
                pltpu.make_async_copy(*o).wait()

    NH = NV // 2

    def group16h(buf, base, off, cur, half):
        cb = half * 256
        v = iv[pl.ds(off, 16)]
        e0 = v[0]
        e15 = v[15]
        fast = jnp.logical_and(e0 == e15, e0 == cur)

        @pl.when(fast)
        def _():
            def frow(k, accs):
                return tuple(accs[j] + buf[base + k, pl.ds(cb + j * 16, 16)]
                             for j in range(NH))
            accs = tuple(areg[pl.ds(cb + j * 16, 16)] for j in range(NH))
            accs = lax.fori_loop(0, 16, frow, accs)
            for j in range(NH):
                areg[pl.ds(cb + j * 16, 16)] = accs[j]

        @pl.when(jnp.logical_not(fast))
        def _():
            def srow(k, carry):
                lcur = carry[0]
                accs = list(carry[1:])
                seg = iv[pl.ds(off + k, 16)][0]
                changed = seg != lcur

                @pl.when(changed)
                def _(lcur=lcur, accs=tuple(accs)):
                    for j in range(NH):
                        areg[pl.ds(cb + j * 16, 16)] = accs[j]

                    @pl.when(lcur == first_seg)
                    def _():
                        pltpu.sync_copy(areg.at[pl.ds(cb, 256)],
                                        bound.at[2 * s, pl.ds(cb, 256)])

                    @pl.when(lcur != first_seg)
                    def _():
                        pltpu.sync_copy(areg.at[pl.ds(cb, 256)],
                                        shared.at[lcur, pl.ds(cb, 256)])

                z = _zvec()
                for j in range(NH):
                    accs[j] = (jnp.where(changed, z, accs[j])
                               + buf[base + k, pl.ds(cb + j * 16, 16)])
                return (seg,) + tuple(accs)

            accs = tuple(areg[pl.ds(cb + j * 16, 16)] for j in range(NH))
            carry = lax.fori_loop(0, 16, srow, (cur,) + accs)
            for j in range(NH):
                areg[pl.ds(cb + j * 16, 16)] = carry[j + 1]

        return e15

    def group16(buf, base, off, cur):
        group16h(buf, base, off, cur, 0)
        return group16h(buf, base, off, cur, 1)

    def compute(buf, ci, cur):
        def group(g, cur):
            return group16(buf, 16 * g, ci * C + 16 * g, cur)
        return lax.fori_loop(0, G, group, cur)

    issue(buf0, sem0, r0)

    @pl.when(s < 15)
    def _():
        pltpu.sync_copy(idx.at[pl.ds(r0, Q)], iv.at[pl.ds(0, Q)])

    @pl.when(s == 15)
    def _():
        pltpu.sync_copy(idx.at[pl.ds(r0, NROWS - 15 * Q)],
                        iv.at[pl.ds(0, NROWS - 15 * Q)])

    first_seg = iv[pl.ds(0, 16)][0]

    def zrow(i, _):
        buf1[i // NV, pl.ds((i % NV) * 16, 16)] = _zvec()
        return 0
    lax.fori_loop(0, 32 * NV, zrow, 0)
    pltpu.sync_copy(buf1.at[pl.ds(0, 32), :], shared.at[pl.ds(s * 32, 32), :])

    for j in range(NV):
        areg[pl.ds(j * 16, 16)] = _zvec()
    plsc.subcore_barrier()

    def pair(t, cur):
        ra = r0 + (2 * t) * C
        rb = ra + C
        issue(buf1, sem1, rb)
        drain(buf0, sem0, ra)
        cur = compute(buf0, 2 * t, cur)

        @pl.when(2 * t + 2 < nch)
        def _():
            issue(buf0, sem0, rb + C)
        drain(buf1, sem1, rb)
        cur = compute(buf1, 2 * t + 1, cur)
        return cur

    cur = lax.fori_loop(0, npairs, pair, first_seg)

    def tail(i, cur):
        drain(buf0, sem0, r0 + i * C)
        return compute(buf0, i, cur)

    cur = lax.fori_loop(2 * npairs, nch, tail, cur)

    def rem16(i, cur):
        r = r0 + nch * C + 16 * i
        ops0, ops1 = dmas(buf0, sem0, r, 16)

        @pl.when(c == 0)
        def _():
            for o in ops0:
                pltpu.sync_copy(o[0], o[1])

        @pl.when(c == 1)
        def _():
            for o in ops1:
                pltpu.sync_copy(o[0], o[1])
        return group16(buf0, 0, nch * C + 16 * i, cur)

    cur = lax.fori_loop(0, (nrows - nch * C) // 16, rem16, cur)

    @pl.when(cur == first_seg)
    def _():
        pltpu.sync_copy(areg, bound.at[2 * s])

    @pl.when(cur != first_seg)
    def _():
        pltpu.sync_copy(areg, bound.at[2 * s + 1])

    plsc.subcore_barrier()

    @pl.when(s == 0)
    def _():
        def addslot(slot, seg):
            pltpu.async_copy(shared.at[seg], t1, sem0)
            pltpu.async_copy(bound.at[slot], t2, sem1)
            pltpu.make_async_copy(shared.at[seg], t1, sem0).wait()
            pltpu.make_async_copy(bound.at[slot], t2, sem1).wait()
            for j in range(NV):
                t1[pl.ds(j * 16, 16)] = (t1[pl.ds(j * 16, 16)]
                                         + t2[pl.ds(j * 16, 16)])
            pltpu.sync_copy(t1, shared.at[seg])

        pltpu.sync_copy(bseg, iv.at[pl.ds(0, 32)])
        vfs = iv[pl.ds(0, 16)]
        vls = iv[pl.ds(16, 16)]
        for w in range(16):
            fs = vfs[w]
            ls = vls[w]
            addslot(2 * w, fs)

            @pl.when(ls != fs)
            def _():
                addslot(2 * w + 1, ls)

    plsc.subcore_barrier()

    @pl.when(c == 0)
    def _():
        pltpu.sync_copy(shared.at[pl.ds(s * 32, 32), :],
                        out.at[pl.ds(s * 32, 32), pl.ds(0, HALF)])

    @pl.when(c == 1)
    def _():
        pltpu.sync_copy(shared.at[pl.ds(s * 32, 32), :],
                        out.at[pl.ds(s * 32, 32), pl.ds(HALF, HALF)])


def _tc_body(idx_ref, h_ref, o_ref):
    i = pl.program_id(0)
    idxb = idx_ref[0, 0, :]
    rows = lax.broadcasted_iota(jnp.int32, (NSEG, 512), 1) + i * 512
    segs = lax.broadcasted_iota(jnp.int32, (NSEG, 512), 0)
    oh = jnp.where((idxb[None, :] == segs) & (rows < NROWS), 1.0, 0.0)
    contrib = jnp.dot(oh.astype(jnp.float32), h_ref[...],
                      preferred_element_type=jnp.float32)

    @pl.when(i == 0)
    def _():
        o_ref[...] = contrib

    @pl.when(i > 0)
    def _():
        o_ref[...] = o_ref[...] + contrib


_NB = (NROWS + 511) // 512


def _tc_matmul(idx3, h2):
    return pl.pallas_call(
        _tc_body,
        grid=(_NB,),
        in_specs=[pl.BlockSpec((1, 1, 512), lambda i: (i, 0, 0)),
                  pl.BlockSpec((512, 512), lambda i: (i, 0))],
        out_specs=pl.BlockSpec((NSEG, 512), lambda i: (0, 0)),
        out_shape=jax.ShapeDtypeStruct((NSEG, 512), jnp.float32),
    )(idx3, h2)


@jax.jit
def kernel(h0, h1, h2, index):
    k = pl.kernel(
        _body,
        out_type=jax.ShapeDtypeStruct((NSEG, 2 * HALF), jnp.float32),
        mesh=plsc.VectorSubcoreMesh(core_axis_name="c", subcore_axis_name="s"),
        scratch_types=[
            pltpu.VMEM((C, HALF), jnp.float32),
            pltpu.VMEM((C, HALF), jnp.float32),
            pltpu.VMEM((Q + 16,), jnp.int32),
            pltpu.VMEM((HALF,), jnp.float32),
            pltpu.VMEM((HALF,), jnp.float32),
            pltpu.VMEM((HALF,), jnp.float32),
            pltpu.VMEM_SHARED((NSEG, HALF), jnp.float32),
            pltpu.VMEM_SHARED((32, HALF), jnp.float32),
            pltpu.SemaphoreType.DMA,
            pltpu.SemaphoreType.DMA,
        ],
    )
    starts = jnp.arange(16, dtype=jnp.int32) * Q
    ends = jnp.minimum(starts + Q, NROWS) - 1
    bseg = jnp.concatenate([index[starts], index[ends]])
    out01 = k(h0, h1, index, bseg)
    idx3 = jnp.pad(index, (0, _NB * 512 - NROWS)).reshape(_NB, 1, 512)
    out2 = _tc_matmul(idx3, h2)
    return jnp.concatenate([out01, out2], axis=1)


# speedup vs baseline: 2.1741x; 1.0009x over previous
import jax
import jax.numpy as jnp
from jax import lax
from jax.experimental import pallas as pl
from jax.experimental.pallas import tpu as pltpu
from jax.experimental.pallas import tpu_sc as plsc

NSEG = 512
NROWS = 50000
HALF = 512
NV = HALF // 16
C = 80
Q = 3200
G = C // 16


def _zvec():
    return jnp.zeros((16,), jnp.float32)


def _body(h0, h1, idx, bseg, out,
          buf0, buf1, iv, areg, t1, t2, shared, bound, sem0, sem1):
    c = lax.axis_index("c")
    s = lax.axis_index("s")

    r0 = s * Q
    nrows = jnp.minimum(Q, NROWS - r0)
    nch = nrows // C
    npairs = nch // 2

    def dmas(buf, sem, r, n):
        ops0 = [(h0.at[pl.ds(r, n), :], buf.at[pl.ds(0, n), :], sem)]
        ops1 = [(h1.at[pl.ds(r, n), pl.ds(0, 256)], buf.at[pl.ds(0, n), pl.ds(0, 256)], sem),
                (h1.at[pl.ds(r, n), pl.ds(256, 256)], buf.at[pl.ds(0, n), pl.ds(256, 256)], sem)]
        return ops0, ops1

    def issue(buf, sem, r):
        ops0, ops1 = dmas(buf, sem, r, C)

        @pl.when(c == 0)
        def _():
            for o in ops0:
                pltpu.async_copy(*o)

        @pl.when(c == 1)
        def _():
            for o in ops1:
                pltpu.async_copy(*o)

    def drain(buf, sem, r):
        ops0, ops1 = dmas(buf, sem, r, C)

        @pl.when(c == 0)
        def _():
            for o in ops0:
                pltpu.make_async_copy(*o).wait()

        @pl.when(c == 1)
        def _():
            for o in ops1:
                pltpu.make_async_copy(*o).wait()

    NH = NV // 2

    def group16h(buf, base, off, cur, half):
        cb = half * 256
        v = iv[pl.ds(off, 16)]
        e0 = v[0]
        e15 = v[15]
        fast = jnp.logical_and(e0 == e15, e0 == cur)

        @pl.when(fast)
        def _():
            def frow(k, accs):
                return tuple(accs[j] + buf[base + k, pl.ds(cb + j * 16, 16)]
                             for j in range(NH))
            accs = tuple(areg[pl.ds(cb + j * 16, 16)] for j in range(NH))
            accs = lax.fori_loop(0, 16, frow, accs)
            for j in range(NH):
                areg[pl.ds(cb + j * 16, 16)] = accs[j]

        @pl.when(jnp.logical_not(fast))
        def _():
            def srow(k, carry):
                lcur = carry[0]
                accs = list(carry[1:])
                seg = iv[pl.ds(off + k, 16)][0]
                changed = seg != lcur

                @pl.when(changed)
                def _(lcur=lcur, accs=tuple(accs)):
                    for j in range(NH):
                        areg[pl.ds(cb + j * 16, 16)] = accs[j]

                    @pl.when(lcur == first_seg)
                    def _():
                        pltpu.sync_copy(areg.at[pl.ds(cb, 256)],
                                        bound.at[2 * s, pl.ds(cb, 256)])

                    @pl.when(lcur != first_seg)
                    def _():
                        pltpu.sync_copy(areg.at[pl.ds(cb, 256)],
                                        shared.at[lcur, pl.ds(cb, 256)])

                z = _zvec()
                for j in range(NH):
                    accs[j] = (jnp.where(changed, z, accs[j])
                               + buf[base + k, pl.ds(cb + j * 16, 16)])
                return (seg,) + tuple(accs)

            accs = tuple(areg[pl.ds(cb + j * 16, 16)] for j in range(NH))
            carry = lax.fori_loop(0, 16, srow, (cur,) + accs)
            for j in range(NH):
                areg[pl.ds(cb + j * 16, 16)] = carry[j + 1]

        return e15

    def group16(buf, base, off, cur):
        group16h(buf, base, off, cur, 0)
        return group16h(buf, base, off, cur, 1)

    def compute(buf, ci, cur):
        def group(g, cur):
            return group16(buf, 16 * g, ci * C + 16 * g, cur)
        return lax.fori_loop(0, G, group, cur)

    issue(buf0, sem0, r0)

    @pl.when(s < 15)
    def _():
        pltpu.sync_copy(idx.at[pl.ds(r0, Q)], iv.at[pl.ds(0, Q)])

    @pl.when(s == 15)
    def _():
        pltpu.sync_copy(idx.at[pl.ds(r0, NROWS - 15 * Q)],
                        iv.at[pl.ds(0, NROWS - 15 * Q)])

    first_seg = iv[pl.ds(0, 16)][0]

    def zrow(i, _):
        buf1[i // NV, pl.ds((i % NV) * 16, 16)] = _zvec()
        return 0
    lax.fori_loop(0, 32 * NV, zrow, 0)
    pltpu.sync_copy(buf1.at[pl.ds(0, 32), :], shared.at[pl.ds(s * 32, 32), :])

    for j in range(NV):
        areg[pl.ds(j * 16, 16)] = _zvec()
    plsc.subcore_barrier()

    def pair(t, cur):
        ra = r0 + (2 * t) * C
        rb = ra + C
        issue(buf1, sem1, rb)
        drain(buf0, sem0, ra)
        cur = compute(buf0, 2 * t, cur)

        @pl.when(2 * t + 2 < nch)
        def _():
            issue(buf0, sem0, rb + C)
        drain(buf1, sem1, rb)
        cur = compute(buf1, 2 * t + 1, cur)
        return cur

    cur = lax.fori_loop(0, npairs, pair, first_seg)

    def tail(i, cur):
        drain(buf0, sem0, r0 + i * C)
        return compute(buf0, i, cur)

    cur = lax.fori_loop(2 * npairs, nch, tail, cur)

    def rem16(i, cur):
        r = r0 + nch * C + 16 * i
        ops0, ops1 = dmas(buf0, sem0, r, 16)

        @pl.when(c == 0)
        def _():
            for o in ops0:
                pltpu.sync_copy(o[0], o[1])

        @pl.when(c == 1)
        def _():
            for o in ops1:
                pltpu.sync_copy(o[0], o[1])
        return group16(buf0, 0, nch * C + 16 * i, cur)

    cur = lax.fori_loop(0, (nrows - nch * C) // 16, rem16, cur)

    @pl.when(cur == first_seg)
    def _():
        pltpu.sync_copy(areg, bound.at[2 * s])

    @pl.when(cur != first_seg)
    def _():
        pltpu.sync_copy(areg, bound.at[2 * s + 1])

    plsc.subcore_barrier()

    @pl.when(s == 0)
    def _():
        def addslot(slot, seg):
            pltpu.async_copy(shared.at[seg], t1, sem0)
            pltpu.async_copy(bound.at[slot], t2, sem1)
            pltpu.make_async_copy(shared.at[seg], t1, sem0).wait()
            pltpu.make_async_copy(bound.at[slot], t2, sem1).wait()
            for j in range(NV):
                t1[pl.ds(j * 16, 16)] = (t1[pl.ds(j * 16, 16)]
                                         + t2[pl.ds(j * 16, 16)])
            pltpu.sync_copy(t1, shared.at[seg])

        pltpu.sync_copy(bseg, iv.at[pl.ds(0, 32)])
        vfs = iv[pl.ds(0, 16)]
        vls = iv[pl.ds(16, 16)]
        for w in range(16):
            fs = vfs[w]
            ls = vls[w]
            addslot(2 * w, fs)

            @pl.when(ls != fs)
            def _():
                addslot(2 * w + 1, ls)

    plsc.subcore_barrier()

    @pl.when(c == 0)
    def _():
        pltpu.sync_copy(shared.at[pl.ds(s * 32, 32), :],
                        out.at[pl.ds(s * 32, 32), pl.ds(0, HALF)])

    @pl.when(c == 1)
    def _():
        pltpu.sync_copy(shared.at[pl.ds(s * 32, 32), :],
                        out.at[pl.ds(s * 32, 32), pl.ds(HALF, HALF)])


def _tc_body(idx_ref, h_ref, o_ref):
    i = pl.program_id(0)
    idxb = idx_ref[0, 0, :]
    rows = lax.broadcasted_iota(jnp.int32, (NSEG, 512), 1) + i * 512
    segs = lax.broadcasted_iota(jnp.int32, (NSEG, 512), 0)
    oh = jnp.where((idxb[None, :] == segs) & (rows < NROWS), 1.0, 0.0)
    contrib = jnp.dot(oh.astype(jnp.float32), h_ref[...],
                      preferred_element_type=jnp.float32)

    @pl.when(i == 0)
    def _():
        o_ref[...] = contrib

    @pl.when(i > 0)
    def _():
        o_ref[...] = o_ref[...] + contrib


_NB = (NROWS + 511) // 512


def _tc_matmul(idx3, h2):
    return pl.pallas_call(
        _tc_body,
        grid=(_NB,),
        in_specs=[pl.BlockSpec((1, 1, 512), lambda i: (i, 0, 0)),
                  pl.BlockSpec((512, 512), lambda i: (i, 0))],
        out_specs=pl.BlockSpec((NSEG, 512), lambda i: (0, 0)),
        out_shape=jax.ShapeDtypeStruct((NSEG, 512), jnp.float32),
    )(idx3, h2)


@jax.jit
def kernel(h0, h1, h2, index):
    k = pl.kernel(
        _body,
        out_type=jax.ShapeDtypeStruct((NSEG, 2 * HALF), jnp.float32),
        mesh=plsc.VectorSubcoreMesh(core_axis_name="c", subcore_axis_name="s"),
        scratch_types=[
            pltpu.VMEM((C, HALF), jnp.float32),
            pltpu.VMEM((C, HALF), jnp.float32),
            pltpu.VMEM((Q + 16,), jnp.int32),
            pltpu.VMEM((HALF,), jnp.float32),
            pltpu.VMEM((HALF,), jnp.float32),
            pltpu.VMEM((HALF,), jnp.float32),
            pltpu.VMEM_SHARED((NSEG, HALF), jnp.float32),
            pltpu.VMEM_SHARED((32, HALF), jnp.float32),
            pltpu.SemaphoreType.DMA,
            pltpu.SemaphoreType.DMA,
        ],
    )
    starts = jnp.arange(16, dtype=jnp.int32) * Q
    ends = jnp.minimum(starts + Q, NROWS) - 1
    bseg = jnp.concatenate([index[starts], index[ends]])
    idx3 = jnp.pad(index, (0, _NB * 512 - NROWS)).reshape(_NB, 1, 512)
    out01 = k(h0, h1, index, bseg)
    out2 = _tc_matmul(idx3, h2)
    return jnp.concatenate([out01, out2], axis=1)
